# baseline (speedup 1.0000x reference)
import functools

import jax
import jax.numpy as jnp
from jax import lax
from jax.experimental import pallas as pl
from jax.experimental.pallas import tpu as pltpu
from jax.experimental.pallas import tpu_sc as plsc

B, L, F, E = 4096, 200, 20, 64
G0, G1 = 104, 96
NC, NS = 2, 16
NW = NC * NS
BW = B // NW
R = 8

_PACK_BB = 32


def _pack_body(sal_ref, idx_ref):
    v = sal_ref[...]
    w = (jnp.int32(1) << jnp.arange(F, dtype=jnp.int32))[None, None, :]
    s = jnp.sum(v * w, axis=2)
    s0 = jnp.pad(s[:, :G0], ((0, 0), (0, 128 - G0)))
    s1 = jnp.pad(s[:, G0:], ((0, 0), (0, 128 - G1)))
    idx_ref[...] = jnp.stack([s0, s1], axis=0)


def _pack_indices(salience_values):
    return pl.pallas_call(
        _pack_body,
        grid=(B // _PACK_BB,),
        in_specs=[pl.BlockSpec((_PACK_BB, L, F), lambda i: (i, 0, 0))],
        out_specs=pl.BlockSpec((2, _PACK_BB, 128), lambda i: (0, i, 0)),
        out_shape=jax.ShapeDtypeStruct((2, B, 128), jnp.int32),
    )(salience_values)


@functools.cache
def _make_gather_kernel():
    mesh = plsc.VectorSubcoreMesh(core_axis_name="c", subcore_axis_name="s")

    @functools.partial(
        pl.kernel,
        mesh=mesh,
        out_type=jax.ShapeDtypeStruct((B, L, E), jnp.float32),
        scratch_types=[
            pltpu.VMEM((2 * BW, 128), jnp.int32),
            pltpu.VMEM((R, G0, E), jnp.float32),
            pltpu.SemaphoreType.DMA,
            pltpu.SemaphoreType.DMA,
        ],
        compiler_params=pltpu.CompilerParams(use_tc_tiling_on_sc=False),
    )
    def _gather_kernel(idx_hbm, table_hbm, out_hbm, idx_v, rows_v, gsem, osem):
        wid = lax.axis_index("s") * NC + lax.axis_index("c")
        b0 = wid * BW
        pltpu.sync_copy(idx_hbm.at[0, pl.ds(b0, BW)], idx_v.at[pl.ds(0, BW)])
        pltpu.sync_copy(idx_hbm.at[1, pl.ds(b0, BW)], idx_v.at[pl.ds(BW, BW)])

        def make_body(gl, l0, trow):
            def body(i, _):
                r0 = i * R
                gets = []
                for k in range(R):
                    gets.append(
                        pltpu.async_copy(
                            table_hbm.at[idx_v.at[trow + r0 + k, pl.ds(0, gl)]],
                            rows_v.at[k, pl.ds(0, gl)],
                            gsem,
                        )
                    )
                puts = []
                for k in range(R):
                    gets[k].wait()
                    puts.append(
                        pltpu.async_copy(
                            rows_v.at[k, pl.ds(0, gl)],
                            out_hbm.at[b0 + r0 + k, pl.ds(l0, gl)],
                            osem,
                        )
                    )
                for p in puts:
                    p.wait()
                return ()

            return body

        lax.fori_loop(0, BW // R, make_body(G0, 0, 0), (), unroll=False)
        lax.fori_loop(0, BW // R, make_body(G1, G0, BW), (), unroll=False)

    return _gather_kernel


def kernel(salience_values, table):
    idx = _pack_indices(salience_values)
    return _make_gather_kernel()(idx, table)


# speedup vs baseline: 1.3241x; 1.3241x over previous
import functools

import jax
import jax.numpy as jnp
from jax import lax
from jax.experimental import pallas as pl
from jax.experimental.pallas import tpu as pltpu
from jax.experimental.pallas import tpu_sc as plsc

B, L, F, E = 4096, 200, 20, 64
G0, G1 = 104, 96
NC, NS = 2, 16
NW = NC * NS
BW = B // NW
R = 8

_PACK_BB = 512


def _pack_body(sal_ref, idx_ref):
    v = sal_ref[...]
    w = (jnp.int32(1) << jnp.arange(F, dtype=jnp.int32))[:, None, None]
    s = jnp.sum(v * w, axis=0)
    s0 = jnp.pad(s[:G0], ((0, 128 - G0), (0, 0)))
    s1 = jnp.pad(s[G0:], ((0, 128 - G1), (0, 0)))
    t0 = jnp.transpose(s0, (1, 0))
    t1 = jnp.transpose(s1, (1, 0))
    idx_ref[...] = jnp.stack([t0, t1], axis=0)


def _pack_indices(salience_values):
    sal_t = jnp.transpose(salience_values, (2, 1, 0))
    return pl.pallas_call(
        _pack_body,
        grid=(B // _PACK_BB,),
        in_specs=[pl.BlockSpec((F, L, _PACK_BB), lambda i: (0, 0, i))],
        out_specs=pl.BlockSpec((2, _PACK_BB, 128), lambda i: (0, i, 0)),
        out_shape=jax.ShapeDtypeStruct((2, B, 128), jnp.int32),
    )(sal_t)


@functools.cache
def _make_gather_kernel():
    mesh = plsc.VectorSubcoreMesh(core_axis_name="c", subcore_axis_name="s")

    @functools.partial(
        pl.kernel,
        mesh=mesh,
        out_type=jax.ShapeDtypeStruct((B, L, E), jnp.float32),
        scratch_types=[
            pltpu.VMEM((2 * BW, 128), jnp.int32),
            pltpu.VMEM((R, G0, E), jnp.float32),
            pltpu.SemaphoreType.DMA,
            pltpu.SemaphoreType.DMA,
        ],
        compiler_params=pltpu.CompilerParams(use_tc_tiling_on_sc=False),
    )
    def _gather_kernel(idx_hbm, table_hbm, out_hbm, idx_v, rows_v, gsem, osem):
        wid = lax.axis_index("s") * NC + lax.axis_index("c")
        b0 = wid * BW
        pltpu.sync_copy(idx_hbm.at[0, pl.ds(b0, BW)], idx_v.at[pl.ds(0, BW)])
        pltpu.sync_copy(idx_hbm.at[1, pl.ds(b0, BW)], idx_v.at[pl.ds(BW, BW)])

        def make_body(gl, l0, trow):
            def body(i, _):
                r0 = i * R
                gets = []
                for k in range(R):
                    gets.append(
                        pltpu.async_copy(
                            table_hbm.at[idx_v.at[trow + r0 + k, pl.ds(0, gl)]],
                            rows_v.at[k, pl.ds(0, gl)],
                            gsem,
                        )
                    )
                puts = []
                for k in range(R):
                    gets[k].wait()
                    puts.append(
                        pltpu.async_copy(
                            rows_v.at[k, pl.ds(0, gl)],
                            out_hbm.at[b0 + r0 + k, pl.ds(l0, gl)],
                            osem,
                        )
                    )
                for p in puts:
                    p.wait()
                return ()

            return body

        lax.fori_loop(0, BW // R, make_body(G0, 0, 0), (), unroll=False)
        lax.fori_loop(0, BW // R, make_body(G1, G0, BW), (), unroll=False)

    return _gather_kernel


def kernel(salience_values, table):
    idx = _pack_indices(salience_values)
    return _make_gather_kernel()(idx, table)
